# baseline (speedup 1.0000x reference)
import functools

import jax
import jax.numpy as jnp
from jax.experimental import pallas as pl

_N = 10000
_E = 160000
_HEADS = 4
_HID = 128
_NG = 20

_N_PAD = 10240
_CH = 1024
_NCH = _N_PAD // _CH
_BE = 512


def _mm_body(x_ref, w_ref, o_ref):
    o_ref[...] = jnp.dot(x_ref[...], w_ref[...],
                         preferred_element_type=jnp.float32)


def _matmul(x, w, bm):
    m, k = x.shape
    n = w.shape[1]
    grid = (m // bm,)
    return pl.pallas_call(
        _mm_body,
        grid=grid,
        in_specs=[
            pl.BlockSpec((bm, k), lambda i: (i, 0)),
            pl.BlockSpec((k, n), lambda i: (0, 0)),
        ],
        out_specs=pl.BlockSpec((bm, n), lambda i: (i, 0)),
        out_shape=jax.ShapeDtypeStruct((m, n), jnp.float32),
    )(x, w)


def _msg_body(si_ref, di_ref, al_ref, h_ref, out_ref):
    j = pl.program_id(1)

    @pl.when(j == 0)
    def _init():
        out_ref[...] = jnp.zeros_like(out_ref)

    si = si_ref[...]
    di = di_ref[...]
    al = al_ref[0]

    acc = jnp.zeros((_BE, _HID), jnp.float32)
    for k in range(_NCH):
        ids = k * _CH + jax.lax.broadcasted_iota(jnp.int32, (_BE, _CH), 1)
        oh = (si == ids).astype(jnp.float32)
        acc = acc + jnp.dot(oh, h_ref[pl.ds(k * _CH, _CH), :],
                            preferred_element_type=jnp.float32)
    w = acc * al
    for k in range(_NCH):
        rids = k * _CH + jax.lax.broadcasted_iota(jnp.int32, (_CH, _BE), 0)
        ohT = (rids == di).astype(jnp.float32)
        out_ref[pl.ds(k * _CH, _CH), :] += jnp.dot(
            ohT, w, preferred_element_type=jnp.float32)


def _message_pass(si_col, di_row, alpha, h_pad, heads):
    e_pad = si_col.shape[0]
    nj = e_pad // _BE
    grid = (heads, nj)
    return pl.pallas_call(
        _msg_body,
        grid=grid,
        in_specs=[
            pl.BlockSpec((_BE, 1), lambda g, j: (j, 0)),
            pl.BlockSpec((1, _BE), lambda g, j: (0, j)),
            pl.BlockSpec((1, _BE, 1), lambda g, j: (g, j, 0)),
            pl.BlockSpec((_N_PAD, _HID), lambda g, j: (0, g)),
        ],
        out_specs=pl.BlockSpec((_N_PAD, _HID), lambda g, j: (0, g)),
        out_shape=jax.ShapeDtypeStruct((_N_PAD, heads * _HID), jnp.float32),
    )(si_col, di_row, alpha, h_pad)


def _edge_alpha(h, a_src, a_dst, si, di, heads):
    n = h.shape[0]
    hh = h.reshape(n, heads, _HID)
    a_s = jnp.sum(hh * a_src[None], axis=-1)
    a_d = jnp.sum(hh * a_dst[None], axis=-1)
    e = jax.nn.leaky_relu(a_s[si] + a_d[di], negative_slope=0.2)
    m = jax.ops.segment_max(e, di, num_segments=n)
    m = jnp.where(jnp.isfinite(m), m, 0.0)
    ex = jnp.exp(e - m[di])
    ss = jax.ops.segment_sum(ex, di, num_segments=n)
    return ex / (ss[di] + 1e-16)


def _gat_layer(h_in_pad, si, di, si_col, di_row, e_pad, W, a_src, a_dst, b,
               heads):
    hm = _matmul(h_in_pad, W, 1024)
    alpha = _edge_alpha(hm[:_N], a_src, a_dst, si, di, heads)
    alpha_pad = jnp.zeros((e_pad, heads), jnp.float32).at[:si.shape[0]].set(alpha)
    alpha_pad = alpha_pad.T.reshape(heads, e_pad, 1)
    out = _message_pass(si_col, di_row, alpha_pad, hm, heads)
    out = jax.nn.relu(out[:_N] + b)
    return jnp.zeros((_N_PAD, heads * _HID), jnp.float32).at[:_N].set(out)


def kernel(x, edge_index, batch, W1, a1_src, a1_dst, b1, W2, a2_src, a2_dst,
           b2, Wc, bc, Wf1, bf1, Wf2, bf2):
    src, dst = edge_index[0], edge_index[1]
    loops = jnp.arange(_N, dtype=edge_index.dtype)
    si = jnp.concatenate([src, loops])
    di = jnp.concatenate([dst, loops])
    e_tot = _E + _N
    e_pad = ((e_tot + _BE - 1) // _BE) * _BE
    fill = jnp.full((e_pad - e_tot,), _N_PAD - 1, jnp.int32)
    si_col = jnp.concatenate([si, fill]).reshape(e_pad, 1)
    di_row = jnp.concatenate([di, fill]).reshape(1, e_pad)

    x_pad = jnp.zeros((_N_PAD, x.shape[1]), jnp.float32).at[:_N].set(x)

    h1 = _gat_layer(x_pad, si, di, si_col, di_row, e_pad,
                    W1, a1_src, a1_dst, b1, _HEADS)
    h2 = _gat_layer(h1, si, di, si_col, di_row, e_pad,
                    W2, a2_src, a2_dst, b2, 1)
    h = h2[:_N]

    clf = jax.nn.softmax(h @ Wc + bc, axis=1)

    bs, bd = batch[src], batch[dst]
    valid = (bs == bd).astype(jnp.float32)
    cs = clf[src] * valid[:, None]
    cd = clf[dst]
    contrib = (cs[:, :, None] * cd[:, None, :]).reshape(_E, 4)
    agg = jax.ops.segment_sum(contrib, bs, num_segments=_NG).reshape(_NG, 2, 2)
    rows = jnp.maximum(jnp.sum(jnp.abs(agg), axis=2, keepdims=True), 1e-5)
    norm = agg / rows
    diag = jnp.stack([norm[:, 0, 0], norm[:, 1, 1]], axis=1)
    loss = jnp.mean(jnp.mean((diag - 1.0) ** 2, axis=1))

    sub_emb = jax.ops.segment_sum(clf[:, 0:1] * h, batch, num_segments=_NG)
    out = jax.nn.relu(sub_emb @ Wf1 + bf1) @ Wf2 + bf2
    return (out, loss)


# speedup vs baseline: 1.6744x; 1.6744x over previous
import functools

import jax
import jax.numpy as jnp
from jax.experimental import pallas as pl

_N = 10000
_E = 160000
_HEADS = 4
_HID = 128
_NG = 20

_N_PAD = 10240
_CH = 1024
_NCH = _N_PAD // _CH
_BE = 512


def _mm_body(x_ref, w_ref, o_ref):
    o_ref[...] = jnp.dot(x_ref[...], w_ref[...],
                         preferred_element_type=jnp.float32)


def _matmul(x, w, bm):
    m, k = x.shape
    n = w.shape[1]
    grid = (m // bm,)
    return pl.pallas_call(
        _mm_body,
        grid=grid,
        in_specs=[
            pl.BlockSpec((bm, k), lambda i: (i, 0)),
            pl.BlockSpec((k, n), lambda i: (0, 0)),
        ],
        out_specs=pl.BlockSpec((bm, n), lambda i: (i, 0)),
        out_shape=jax.ShapeDtypeStruct((m, n), jnp.float32),
    )(x, w)


def _msg_body(heads, si_ref, di_ref, al_ref, h_ref, out_ref):
    j = pl.program_id(0)

    @pl.when(j == 0)
    def _init():
        out_ref[...] = jnp.zeros_like(out_ref)

    si = si_ref[...]
    di = di_ref[...]
    al = al_ref[...]

    f = heads * _HID
    acc = jnp.zeros((_BE, f), jnp.float32)
    for k in range(_NCH):
        ids = k * _CH + jax.lax.broadcasted_iota(jnp.int32, (_BE, _CH), 1)
        oh = (si == ids).astype(jnp.bfloat16)
        acc = acc + jnp.dot(oh, h_ref[pl.ds(k * _CH, _CH), :],
                            preferred_element_type=jnp.float32)
    if heads == 1:
        w = acc * al
    else:
        w = jnp.concatenate(
            [acc[:, g * _HID:(g + 1) * _HID] * al[:, g:g + 1]
             for g in range(heads)], axis=1)
    wb = w.astype(jnp.bfloat16)
    for k in range(_NCH):
        rids = k * _CH + jax.lax.broadcasted_iota(jnp.int32, (_CH, _BE), 0)
        ohT = (rids == di).astype(jnp.bfloat16)
        out_ref[pl.ds(k * _CH, _CH), :] += jnp.dot(
            ohT, wb, preferred_element_type=jnp.float32)


def _message_pass(si_col, di_row, alpha, h_pad, heads):
    e_pad = si_col.shape[0]
    nj = e_pad // _BE
    f = heads * _HID
    return pl.pallas_call(
        functools.partial(_msg_body, heads),
        grid=(nj,),
        in_specs=[
            pl.BlockSpec((_BE, 1), lambda j: (j, 0)),
            pl.BlockSpec((1, _BE), lambda j: (0, j)),
            pl.BlockSpec((_BE, heads), lambda j: (j, 0)),
            pl.BlockSpec((_N_PAD, f), lambda j: (0, 0)),
        ],
        out_specs=pl.BlockSpec((_N_PAD, f), lambda j: (0, 0)),
        out_shape=jax.ShapeDtypeStruct((_N_PAD, f), jnp.float32),
    )(si_col, di_row, alpha, h_pad)


def _edge_alpha(h, a_src, a_dst, si, di, heads):
    n = h.shape[0]
    hh = h.reshape(n, heads, _HID)
    a_s = jnp.sum(hh * a_src[None], axis=-1)
    a_d = jnp.sum(hh * a_dst[None], axis=-1)
    e = jax.nn.leaky_relu(a_s[si] + a_d[di], negative_slope=0.2)
    m = jax.ops.segment_max(e, di, num_segments=n)
    m = jnp.where(jnp.isfinite(m), m, 0.0)
    ex = jnp.exp(e - m[di])
    ss = jax.ops.segment_sum(ex, di, num_segments=n)
    return ex / (ss[di] + 1e-16)


def _gat_layer(h_in_pad, si, di, si_col, di_row, e_pad, W, a_src, a_dst, b,
               heads):
    hm = _matmul(h_in_pad, W, 1024)
    alpha = _edge_alpha(hm[:_N], a_src, a_dst, si, di, heads)
    alpha_pad = jnp.zeros((e_pad, heads), jnp.float32).at[:si.shape[0]].set(alpha)
    out = _message_pass(si_col, di_row, alpha_pad, hm.astype(jnp.bfloat16),
                        heads)
    out = jax.nn.relu(out[:_N] + b)
    return jnp.zeros((_N_PAD, heads * _HID), jnp.float32).at[:_N].set(out)


def kernel(x, edge_index, batch, W1, a1_src, a1_dst, b1, W2, a2_src, a2_dst,
           b2, Wc, bc, Wf1, bf1, Wf2, bf2):
    src, dst = edge_index[0], edge_index[1]
    loops = jnp.arange(_N, dtype=edge_index.dtype)
    si = jnp.concatenate([src, loops])
    di = jnp.concatenate([dst, loops])
    e_tot = _E + _N
    e_pad = ((e_tot + _BE - 1) // _BE) * _BE
    fill = jnp.full((e_pad - e_tot,), _N_PAD - 1, jnp.int32)
    si_col = jnp.concatenate([si, fill]).reshape(e_pad, 1)
    di_row = jnp.concatenate([di, fill]).reshape(1, e_pad)

    x_pad = jnp.zeros((_N_PAD, x.shape[1]), jnp.float32).at[:_N].set(x)

    h1 = _gat_layer(x_pad, si, di, si_col, di_row, e_pad,
                    W1, a1_src, a1_dst, b1, _HEADS)
    h2 = _gat_layer(h1, si, di, si_col, di_row, e_pad,
                    W2, a2_src, a2_dst, b2, 1)
    h = h2[:_N]

    clf = jax.nn.softmax(h @ Wc + bc, axis=1)

    bs, bd = batch[src], batch[dst]
    valid = (bs == bd).astype(jnp.float32)
    cs = clf[src] * valid[:, None]
    cd = clf[dst]
    contrib = (cs[:, :, None] * cd[:, None, :]).reshape(_E, 4)
    agg = jax.ops.segment_sum(contrib, bs, num_segments=_NG).reshape(_NG, 2, 2)
    rows = jnp.maximum(jnp.sum(jnp.abs(agg), axis=2, keepdims=True), 1e-5)
    norm = agg / rows
    diag = jnp.stack([norm[:, 0, 0], norm[:, 1, 1]], axis=1)
    loss = jnp.mean(jnp.mean((diag - 1.0) ** 2, axis=1))

    sub_emb = jax.ops.segment_sum(clf[:, 0:1] * h, batch, num_segments=_NG)
    out = jax.nn.relu(sub_emb @ Wf1 + bf1) @ Wf2 + bf2
    return (out, loss)


# speedup vs baseline: 1.8595x; 1.1105x over previous
import functools

import jax
import jax.numpy as jnp
from jax.experimental import pallas as pl

_N = 10000
_E = 160000
_HEADS = 4
_HID = 128
_NG = 20

_N_PAD = 10240
_CH = 1024
_NCH = _N_PAD // _CH
_BE = 512


def _mm_body(x_ref, w_ref, o_ref):
    o_ref[...] = jnp.dot(x_ref[...], w_ref[...],
                         preferred_element_type=jnp.float32)


def _matmul(x, w, bm):
    m, k = x.shape
    n = w.shape[1]
    grid = (m // bm,)
    return pl.pallas_call(
        _mm_body,
        grid=grid,
        in_specs=[
            pl.BlockSpec((bm, k), lambda i: (i, 0)),
            pl.BlockSpec((k, n), lambda i: (0, 0)),
        ],
        out_specs=pl.BlockSpec((bm, n), lambda i: (i, 0)),
        out_shape=jax.ShapeDtypeStruct((m, n), jnp.float32),
    )(x, w)


def _msg_body(heads, si_ref, di_ref, al_ref, h_ref, out_ref, ss_ref):
    j = pl.program_id(0)

    @pl.when(j == 0)
    def _init():
        out_ref[...] = jnp.zeros_like(out_ref)
        ss_ref[...] = jnp.zeros_like(ss_ref)

    si = si_ref[...]
    di = di_ref[...]
    al = al_ref[...]

    f = heads * _HID
    acc = jnp.zeros((_BE, f), jnp.float32)
    for k in range(_NCH):
        ids = k * _CH + jax.lax.broadcasted_iota(jnp.int32, (_BE, _CH), 1)
        oh = (si == ids).astype(jnp.bfloat16)
        acc = acc + jnp.dot(oh, h_ref[pl.ds(k * _CH, _CH), :],
                            preferred_element_type=jnp.float32)
    if heads == 1:
        w = acc * al
    else:
        w = jnp.concatenate(
            [acc[:, g * _HID:(g + 1) * _HID] * al[:, g:g + 1]
             for g in range(heads)], axis=1)
    wb = w.astype(jnp.bfloat16)
    alb = al.astype(jnp.bfloat16)
    for k in range(_NCH):
        rids = k * _CH + jax.lax.broadcasted_iota(jnp.int32, (_CH, _BE), 0)
        ohT = (rids == di).astype(jnp.bfloat16)
        out_ref[pl.ds(k * _CH, _CH), :] += jnp.dot(
            ohT, wb, preferred_element_type=jnp.float32)
        ss_ref[pl.ds(k * _CH, _CH), :] += jnp.dot(
            ohT, alb, preferred_element_type=jnp.float32)


def _message_pass(si_col, di_row, alpha, h_pad, heads):
    e_pad = si_col.shape[0]
    nj = e_pad // _BE
    f = heads * _HID
    return pl.pallas_call(
        functools.partial(_msg_body, heads),
        grid=(nj,),
        in_specs=[
            pl.BlockSpec((_BE, 1), lambda j: (j, 0)),
            pl.BlockSpec((1, _BE), lambda j: (0, j)),
            pl.BlockSpec((_BE, heads), lambda j: (j, 0)),
            pl.BlockSpec((_N_PAD, f), lambda j: (0, 0)),
        ],
        out_specs=[
            pl.BlockSpec((_N_PAD, f), lambda j: (0, 0)),
            pl.BlockSpec((_N_PAD, heads), lambda j: (0, 0)),
        ],
        out_shape=[
            jax.ShapeDtypeStruct((_N_PAD, f), jnp.float32),
            jax.ShapeDtypeStruct((_N_PAD, heads), jnp.float32),
        ],
    )(si_col, di_row, alpha, h_pad)


def _edge_ex(h, a_src, a_dst, si, di, heads):
    n = h.shape[0]
    hh = h.reshape(n, heads, _HID)
    a_s = jnp.sum(hh * a_src[None], axis=-1)
    a_d = jnp.sum(hh * a_dst[None], axis=-1)
    e = jax.nn.leaky_relu(a_s[si] + a_d[di], negative_slope=0.2)
    return jnp.exp(e - jnp.max(e))


def _gat_layer(h_in_pad, si, di, si_col, di_row, e_pad, W, a_src, a_dst, b,
               heads):
    hm = _matmul(h_in_pad, W, 1024)
    ex = _edge_ex(hm[:_N], a_src, a_dst, si, di, heads)
    ex_pad = jnp.zeros((e_pad, heads), jnp.float32).at[:si.shape[0]].set(ex)
    num, ss = _message_pass(si_col, di_row, ex_pad, hm.astype(jnp.bfloat16),
                            heads)
    num = num[:_N].reshape(_N, heads, _HID)
    out = (num / (ss[:_N, :, None] + 1e-16)).reshape(_N, heads * _HID)
    out = jax.nn.relu(out + b)
    return jnp.zeros((_N_PAD, heads * _HID), jnp.float32).at[:_N].set(out)


def kernel(x, edge_index, batch, W1, a1_src, a1_dst, b1, W2, a2_src, a2_dst,
           b2, Wc, bc, Wf1, bf1, Wf2, bf2):
    src, dst = edge_index[0], edge_index[1]
    loops = jnp.arange(_N, dtype=edge_index.dtype)
    si = jnp.concatenate([src, loops])
    di = jnp.concatenate([dst, loops])
    e_tot = _E + _N
    e_pad = ((e_tot + _BE - 1) // _BE) * _BE
    fill = jnp.full((e_pad - e_tot,), _N_PAD - 1, jnp.int32)
    si_col = jnp.concatenate([si, fill]).reshape(e_pad, 1)
    di_row = jnp.concatenate([di, fill]).reshape(1, e_pad)

    x_pad = jnp.zeros((_N_PAD, x.shape[1]), jnp.float32).at[:_N].set(x)

    h1 = _gat_layer(x_pad, si, di, si_col, di_row, e_pad,
                    W1, a1_src, a1_dst, b1, _HEADS)
    h2 = _gat_layer(h1, si, di, si_col, di_row, e_pad,
                    W2, a2_src, a2_dst, b2, 1)
    h = h2[:_N]

    clf = jax.nn.softmax(h @ Wc + bc, axis=1)

    bs, bd = batch[src], batch[dst]
    valid = (bs == bd).astype(jnp.float32)
    cs = clf[src] * valid[:, None]
    cd = clf[dst]
    contrib = (cs[:, :, None] * cd[:, None, :]).reshape(_E, 4)
    agg = jax.ops.segment_sum(contrib, bs, num_segments=_NG).reshape(_NG, 2, 2)
    rows = jnp.maximum(jnp.sum(jnp.abs(agg), axis=2, keepdims=True), 1e-5)
    norm = agg / rows
    diag = jnp.stack([norm[:, 0, 0], norm[:, 1, 1]], axis=1)
    loss = jnp.mean(jnp.mean((diag - 1.0) ** 2, axis=1))

    sub_emb = jax.ops.segment_sum(clf[:, 0:1] * h, batch, num_segments=_NG)
    out = jax.nn.relu(sub_emb @ Wf1 + bf1) @ Wf2 + bf2
    return (out, loss)
